# baseline (speedup 1.0000x reference)
import functools

import jax
import jax.numpy as jnp
from jax import lax
from jax.experimental import pallas as pl
from jax.experimental.pallas import tpu as pltpu
from jax.experimental.pallas import tpu_sc as plsc

M = 300
NUM_CLASSES = 80
BOX_DIM = 4
OUT_W = NUM_CLASSES * BOX_DIM
STRIPE = 24
FULL_WORKERS = M // STRIPE
TAIL8_BASE = FULL_WORKERS * STRIPE
TAIL4_BASE = TAIL8_BASE + 8
LANES = 16
CHUNKS = OUT_W // LANES
NUM_LABELS = 8


def _sc_body(w_hbm, labels_hbm, out_w_hbm,
             labels_v, mask_tab, w_v, out_w_v,
             sem_lab, sem_w, sem_out):
    wid = lax.axis_index("s")
    iota = lax.broadcasted_iota(jnp.int32, (LANES,), 0)
    iota4 = iota & 3
    zeros = jnp.zeros((LANES,), jnp.float32)
    ones = jnp.ones((LANES,), jnp.float32)

    lab_cp = pltpu.async_copy(labels_hbm, labels_v, sem_lab)

    @pl.when(wid * STRIPE < NUM_CLASSES)
    def _():
        pltpu.async_copy(w_hbm.at[pl.ds(wid * STRIPE, STRIPE)],
                         w_v, sem_w)

    for i in range(NUM_CLASSES // LANES):
        mask_tab[pl.ds(i * LANES, LANES)] = zeros
    lab_cp.wait()
    lab_vec = plsc.load_gather(labels_v, [iota & 7])
    plsc.store_scatter(mask_tab, [lab_vec], ones)

    def process(base, nrows):
        rows = pl.ds(0, nrows)

        def zero_row(r, carry):
            for v in range(CHUNKS):
                out_w_v[r, pl.ds(v * LANES, LANES)] = zeros
            return carry

        lax.fori_loop(0, nrows, zero_row, 0)

        @pl.when(base < NUM_CLASSES)
        def _():
            pltpu.make_async_copy(w_hbm.at[pl.ds(base, nrows)],
                                  w_v.at[rows], sem_w).wait()

            def diag_row(r, carry):
                r16 = jnp.full((LANES,), r, jnp.int32)
                rg16 = r16 + base
                r_eff16 = jnp.minimum(rg16, NUM_CLASSES - 1)
                w_row = plsc.load_gather(w_v, [r16, iota4])
                mval = plsc.load_gather(mask_tab, [r_eff16])
                col = r_eff16 * BOX_DIM + iota4
                lane_mask = (iota < BOX_DIM) & (rg16 < NUM_CLASSES)
                plsc.store_scatter(out_w_v, [r16, col], w_row * mval,
                                   mask=lane_mask)
                return carry

            lax.fori_loop(0, nrows, diag_row, 0)

        pltpu.async_copy(out_w_v.at[rows],
                         out_w_hbm.at[pl.ds(base, nrows)], sem_out).wait()

    @pl.when(wid < FULL_WORKERS)
    def _():
        process(wid * STRIPE, STRIPE)

    @pl.when(wid == FULL_WORKERS)
    def _():
        process(TAIL8_BASE, 8)

    @pl.when(wid == FULL_WORKERS + 1)
    def _():
        process(TAIL4_BASE, 4)


def _tc_body(labels_smem, t_ref, out_ref):
    col = lax.broadcasted_iota(jnp.int32, (1, OUT_W), 1)
    cls = col >> 2
    box = col & 3
    mask = cls == labels_smem[0]
    for k in range(1, NUM_LABELS):
        mask = mask | (cls == labels_smem[k])
    t = t_ref[...]
    tt = jnp.where(
        box == 0, t[:, 0:1],
        jnp.where(box == 1, t[:, 1:2],
                  jnp.where(box == 2, t[:, 2:3], t[:, 3:4])))
    out_ref[...] = jnp.where(mask, tt, jnp.float32(0.0))


@jax.jit
def kernel(bbox_targets, bbox_weights, labels):
    out_t = pl.pallas_call(
        _tc_body,
        out_shape=jax.ShapeDtypeStruct((M, OUT_W), jnp.float32),
        in_specs=[pl.BlockSpec(memory_space=pltpu.SMEM),
                  pl.BlockSpec(memory_space=pltpu.VMEM)],
        out_specs=pl.BlockSpec(memory_space=pltpu.VMEM),
    )(labels, bbox_targets)

    mesh = plsc.VectorSubcoreMesh(core_axis_name="c", subcore_axis_name="s",
                                  num_cores=1)
    out_w = pl.kernel(
        _sc_body,
        out_type=jax.ShapeDtypeStruct((M, OUT_W), jnp.float32),
        mesh=mesh,
        compiler_params=pltpu.CompilerParams(use_tc_tiling_on_sc=True,
                                             needs_layout_passes=False,
                                             skip_device_barrier=True),
        scratch_types=[
            pltpu.VMEM((NUM_LABELS,), jnp.int32),
            pltpu.VMEM((NUM_CLASSES,), jnp.float32),
            pltpu.VMEM((STRIPE, BOX_DIM), jnp.float32),
            pltpu.VMEM((STRIPE, OUT_W), jnp.float32),
            pltpu.SemaphoreType.DMA,
            pltpu.SemaphoreType.DMA,
            pltpu.SemaphoreType.DMA,
        ],
    )(bbox_weights, labels)
    return (out_t, out_w)
